# baseline (speedup 1.0000x reference)
import jax
import jax.numpy as jnp
import numpy as np
from jax.experimental import pallas as pl
from jax.experimental.pallas import tpu as pltpu

_B = 128
_V = 100000
_BC = 2048
_SUB = 128
_NBLK = (_V + _BC - 1) // _BC

_KS0 = np.uint32(0)
_KS1 = np.uint32(42)
_KS2 = np.uint32(np.uint32(0) ^ np.uint32(42) ^ np.uint32(0x1BD11BDA))

_TINY = np.float32(np.finfo(np.float32).tiny)
_NEG_INF = np.float32(-np.inf)
_BIG_IDX = np.int32(0x7FFFFFFF)


def _rotl(x, d):
    return jax.lax.shift_left(x, np.uint32(d)) | jax.lax.shift_right_logical(
        x, np.uint32(32 - d)
    )


def _threefry_bits(x1):
    rot_a = (13, 15, 26, 6)
    rot_b = (17, 29, 16, 24)

    def rounds(x0, x1, rots):
        for r in rots:
            x0 = x0 + x1
            x1 = _rotl(x1, r)
            x1 = x1 ^ x0
        return x0, x1

    x0 = x1
    x1 = _rotl(x1, 13) ^ x0
    x0, x1 = rounds(x0, x1, (15, 26, 6))
    x0 = x0 + _KS1
    x1 = x1 + np.uint32(_KS2 + np.uint32(1))
    x0, x1 = rounds(x0, x1, rot_b)
    x0 = x0 + _KS2
    x1 = x1 + np.uint32(_KS0 + np.uint32(2))
    x0, x1 = rounds(x0, x1, rot_a)
    x0 = x0 + _KS0
    x1 = x1 + np.uint32(_KS1 + np.uint32(3))
    x0, x1 = rounds(x0, x1, rot_b)
    x0 = x0 + _KS1
    x1 = x1 + np.uint32(_KS2 + np.uint32(4))
    x0, x1 = rounds(x0, x1, rot_a)
    x0 = x0 + _KS2
    x1 = x1 + np.uint32(_KS0 + np.uint32(5))
    return x0 ^ x1


def _body(hbm_ref, tail_ref, out_ref, buf_ref, flat_ref, rv_ref, ri_ref, sem_ref):
    lane = jax.lax.broadcasted_iota(jnp.int32, (_B, _BC), 1)
    row = jax.lax.broadcasted_iota(jnp.int32, (_B, _BC), 0)
    flat_ref[...] = row * _V + lane + jnp.int32(_KS1)
    rv_ref[...] = jnp.full((_B, _BC), _NEG_INF, jnp.float32)
    ri_ref[...] = jnp.zeros((_B, _BC), jnp.int32)

    def base_of(j):
        return jnp.minimum(j * _BC, _V - _BC)

    def start_copy(j, slot):
        @pl.when(j < _NBLK - 1)
        def _():
            pltpu.make_async_copy(
                hbm_ref.at[:, pl.ds(j * _BC, _BC)],
                buf_ref.at[slot],
                sem_ref.at[slot],
            ).start()

        @pl.when(j == _NBLK - 1)
        def _():
            pltpu.make_async_copy(
                tail_ref, buf_ref.at[slot], sem_ref.at[slot]
            ).start()

    def wait_copy(slot):
        pltpu.make_async_copy(
            hbm_ref.at[:, pl.ds(0, _BC)],
            buf_ref.at[slot],
            sem_ref.at[slot],
        ).wait()

    start_copy(0, 0)
    start_copy(1, 1)

    def step(j, carry):
        slot = jax.lax.rem(j, 3)

        @pl.when(j + 2 < _NBLK)
        def _():
            start_copy(j + 2, jax.lax.rem(j + 2, 3))

        wait_copy(slot)
        base = base_of(j)
        for s in range(_BC // _SUB):
            cs = slice(s * _SUB, (s + 1) * _SUB)
            x = buf_ref[slot, :, cs]
            f42 = (flat_ref[:, cs] + base).astype(jnp.uint32)

            bits = _threefry_bits(f42)
            fbits = jax.lax.shift_right_logical(bits, np.uint32(9)) | np.uint32(
                0x3F800000
            )
            flt = jax.lax.bitcast_convert_type(fbits, jnp.float32) - np.float32(1.0)
            u = jnp.maximum(flt, _TINY)
            val = x - jnp.log(-jnp.log(u))

            better = val > rv_ref[:, cs]
            rv_ref[:, cs] = jnp.where(better, val, rv_ref[:, cs])
            ri_ref[:, cs] = jnp.where(better, f42.astype(jnp.int32), ri_ref[:, cs])
        return 0

    jax.lax.fori_loop(0, _NBLK, step, 0, unroll=7)

    rv = rv_ref[...]
    col = (
        ri_ref[...]
        - jnp.int32(_KS1)
        - jax.lax.broadcasted_iota(jnp.int32, (_B, _BC), 0) * _V
    )
    bm = jnp.max(rv, axis=1, keepdims=True)
    bi = jnp.min(jnp.where(rv == bm, col, _BIG_IDX), axis=1, keepdims=True)
    out_ref[...] = bi


def kernel(logits):
    tail = jax.lax.slice(logits, (0, _V - _BC), (_B, _V))
    out = pl.pallas_call(
        _body,
        in_specs=[
            pl.BlockSpec(memory_space=pl.ANY),
            pl.BlockSpec(memory_space=pl.ANY),
        ],
        out_specs=pl.BlockSpec((_B, 1), memory_space=pltpu.VMEM),
        out_shape=jax.ShapeDtypeStruct((_B, 1), jnp.int32),
        scratch_shapes=[
            pltpu.VMEM((3, _B, _BC), jnp.float32),
            pltpu.VMEM((_B, _BC), jnp.int32),
            pltpu.VMEM((_B, _BC), jnp.float32),
            pltpu.VMEM((_B, _BC), jnp.int32),
            pltpu.SemaphoreType.DMA((3,)),
        ],
    )(logits, tail)
    return out.reshape(_B)


# speedup vs baseline: 1.0039x; 1.0039x over previous
import jax
import jax.numpy as jnp
import numpy as np
from jax.experimental import pallas as pl
from jax.experimental.pallas import tpu as pltpu

_B = 128
_V = 100000
_BC = 2048
_SUB = 256
_NBLK = (_V + _BC - 1) // _BC

_KS0 = np.uint32(0)
_KS1 = np.uint32(42)
_KS2 = np.uint32(np.uint32(0) ^ np.uint32(42) ^ np.uint32(0x1BD11BDA))

_TINY = np.float32(np.finfo(np.float32).tiny)
_NEG_INF = np.float32(-np.inf)
_BIG_IDX = np.int32(0x7FFFFFFF)


def _rotl(x, d):
    return jax.lax.shift_left(x, np.uint32(d)) | jax.lax.shift_right_logical(
        x, np.uint32(32 - d)
    )


def _threefry_bits(x1):
    rot_a = (13, 15, 26, 6)
    rot_b = (17, 29, 16, 24)

    def rounds(x0, x1, rots):
        for r in rots:
            x0 = x0 + x1
            x1 = _rotl(x1, r)
            x1 = x1 ^ x0
        return x0, x1

    x0 = x1
    x1 = _rotl(x1, 13) ^ x0
    x0, x1 = rounds(x0, x1, (15, 26, 6))
    x0 = x0 + _KS1
    x1 = x1 + np.uint32(_KS2 + np.uint32(1))
    x0, x1 = rounds(x0, x1, rot_b)
    x0 = x0 + _KS2
    x1 = x1 + np.uint32(_KS0 + np.uint32(2))
    x0, x1 = rounds(x0, x1, rot_a)
    x0 = x0 + _KS0
    x1 = x1 + np.uint32(_KS1 + np.uint32(3))
    x0, x1 = rounds(x0, x1, rot_b)
    x0 = x0 + _KS1
    x1 = x1 + np.uint32(_KS2 + np.uint32(4))
    x0, x1 = rounds(x0, x1, rot_a)
    x0 = x0 + _KS2
    x1 = x1 + np.uint32(_KS0 + np.uint32(5))
    return x0 ^ x1


def _body(hbm_ref, tail_ref, out_ref, buf_ref, flat_ref, rv_ref, ri_ref, sem_ref):
    lane = jax.lax.broadcasted_iota(jnp.int32, (_B, _BC), 1)
    row = jax.lax.broadcasted_iota(jnp.int32, (_B, _BC), 0)
    flat_ref[...] = row * _V + lane + jnp.int32(_KS1)
    rv_ref[...] = jnp.full((_B, _BC), _NEG_INF, jnp.float32)
    ri_ref[...] = jnp.zeros((_B, _BC), jnp.int32)

    def base_of(j):
        return jnp.minimum(j * _BC, _V - _BC)

    def start_copy(j, slot):
        @pl.when(j < _NBLK - 1)
        def _():
            pltpu.make_async_copy(
                hbm_ref.at[:, pl.ds(j * _BC, _BC)],
                buf_ref.at[slot],
                sem_ref.at[slot],
            ).start()

        @pl.when(j == _NBLK - 1)
        def _():
            pltpu.make_async_copy(
                tail_ref, buf_ref.at[slot], sem_ref.at[slot]
            ).start()

    def wait_copy(slot):
        pltpu.make_async_copy(
            hbm_ref.at[:, pl.ds(0, _BC)],
            buf_ref.at[slot],
            sem_ref.at[slot],
        ).wait()

    start_copy(0, 0)
    start_copy(1, 1)

    def step(j, carry):
        slot = jax.lax.rem(j, 3)

        @pl.when(j + 2 < _NBLK)
        def _():
            start_copy(j + 2, jax.lax.rem(j + 2, 3))

        wait_copy(slot)
        base = base_of(j)
        for s in range(_BC // _SUB):
            cs = slice(s * _SUB, (s + 1) * _SUB)
            x = buf_ref[slot, :, cs]
            f42 = (flat_ref[:, cs] + base).astype(jnp.uint32)

            bits = _threefry_bits(f42)
            fbits = jax.lax.shift_right_logical(bits, np.uint32(9)) | np.uint32(
                0x3F800000
            )
            flt = jax.lax.bitcast_convert_type(fbits, jnp.float32) - np.float32(1.0)
            u = jnp.maximum(flt, _TINY)
            val = x - jnp.log(-jnp.log(u))

            better = val > rv_ref[:, cs]
            rv_ref[:, cs] = jnp.where(better, val, rv_ref[:, cs])
            ri_ref[:, cs] = jnp.where(better, f42.astype(jnp.int32), ri_ref[:, cs])
        return 0

    jax.lax.fori_loop(0, _NBLK, step, 0, unroll=2)

    rv = rv_ref[...]
    col = (
        ri_ref[...]
        - jnp.int32(_KS1)
        - jax.lax.broadcasted_iota(jnp.int32, (_B, _BC), 0) * _V
    )
    bm = jnp.max(rv, axis=1, keepdims=True)
    bi = jnp.min(jnp.where(rv == bm, col, _BIG_IDX), axis=1, keepdims=True)
    out_ref[...] = bi


def kernel(logits):
    tail = jax.lax.slice(logits, (0, _V - _BC), (_B, _V))
    out = pl.pallas_call(
        _body,
        in_specs=[
            pl.BlockSpec(memory_space=pl.ANY),
            pl.BlockSpec(memory_space=pl.ANY),
        ],
        out_specs=pl.BlockSpec((_B, 1), memory_space=pltpu.VMEM),
        out_shape=jax.ShapeDtypeStruct((_B, 1), jnp.int32),
        scratch_shapes=[
            pltpu.VMEM((3, _B, _BC), jnp.float32),
            pltpu.VMEM((_B, _BC), jnp.int32),
            pltpu.VMEM((_B, _BC), jnp.float32),
            pltpu.VMEM((_B, _BC), jnp.int32),
            pltpu.SemaphoreType.DMA((3,)),
        ],
    )(logits, tail)
    return out.reshape(_B)


# speedup vs baseline: 1.0070x; 1.0031x over previous
import jax
import jax.numpy as jnp
import numpy as np
from jax.experimental import pallas as pl
from jax.experimental.pallas import tpu as pltpu

_B = 128
_V = 100000
_BC = 2048
_SUB = 128
_NBLK = (_V + _BC - 1) // _BC

_KS0 = np.uint32(0)
_KS1 = np.uint32(42)
_KS2 = np.uint32(np.uint32(0) ^ np.uint32(42) ^ np.uint32(0x1BD11BDA))

_TINY = np.float32(np.finfo(np.float32).tiny)
_NEG_INF = np.float32(-np.inf)
_BIG_IDX = np.int32(0x7FFFFFFF)


def _rotl(x, d):
    return jax.lax.shift_left(x, np.uint32(d)) | jax.lax.shift_right_logical(
        x, np.uint32(32 - d)
    )


def _threefry_bits(x1):
    rot_a = (13, 15, 26, 6)
    rot_b = (17, 29, 16, 24)

    def rounds(x0, x1, rots):
        for r in rots:
            x0 = x0 + x1
            x1 = _rotl(x1, r)
            x1 = x1 ^ x0
        return x0, x1

    x0 = x1
    x1 = _rotl(x1, 13) ^ x0
    x0, x1 = rounds(x0, x1, (15, 26, 6))
    x0 = x0 + _KS1
    x1 = x1 + np.uint32(_KS2 + np.uint32(1))
    x0, x1 = rounds(x0, x1, rot_b)
    x0 = x0 + _KS2
    x1 = x1 + np.uint32(_KS0 + np.uint32(2))
    x0, x1 = rounds(x0, x1, rot_a)
    x0 = x0 + _KS0
    x1 = x1 + np.uint32(_KS1 + np.uint32(3))
    x0, x1 = rounds(x0, x1, rot_b)
    x0 = x0 + _KS1
    x1 = x1 + np.uint32(_KS2 + np.uint32(4))
    x0, x1 = rounds(x0, x1, rot_a)
    x0 = x0 + _KS2
    x1 = x1 + np.uint32(_KS0 + np.uint32(5))
    return x0 ^ x1


def _body(hbm_ref, tail_ref, out_ref, buf_ref, flat_ref, rv_ref, ri_ref, sem_ref):
    lane = jax.lax.broadcasted_iota(jnp.int32, (_B, _BC), 1)
    row = jax.lax.broadcasted_iota(jnp.int32, (_B, _BC), 0)
    flat_ref[...] = row * _V + lane + jnp.int32(_KS1)
    rv_ref[...] = jnp.full((_B, _BC), _NEG_INF, jnp.float32)
    ri_ref[...] = jnp.zeros((_B, _BC), jnp.int32)

    def base_of(j):
        return jnp.minimum(j * _BC, _V - _BC)

    def start_copy(j, slot):
        @pl.when(j < _NBLK - 1)
        def _():
            pltpu.make_async_copy(
                hbm_ref.at[:, pl.ds(j * _BC, _BC)],
                buf_ref.at[slot],
                sem_ref.at[slot],
            ).start()

        @pl.when(j == _NBLK - 1)
        def _():
            pltpu.make_async_copy(
                tail_ref, buf_ref.at[slot], sem_ref.at[slot]
            ).start()

    def wait_copy(slot):
        pltpu.make_async_copy(
            hbm_ref.at[:, pl.ds(0, _BC)],
            buf_ref.at[slot],
            sem_ref.at[slot],
        ).wait()

    start_copy(0, 0)
    start_copy(1, 1)

    def step(j, carry):
        slot = jax.lax.rem(j, 3)

        @pl.when(j + 2 < _NBLK)
        def _():
            start_copy(j + 2, jax.lax.rem(j + 2, 3))

        wait_copy(slot)
        base = base_of(j)
        for s in range(_BC // _SUB):
            cs = slice(s * _SUB, (s + 1) * _SUB)
            x = buf_ref[slot, :, cs]
            f42 = (flat_ref[:, cs] + base).astype(jnp.uint32)

            bits = _threefry_bits(f42)
            fbits = jax.lax.shift_right_logical(bits, np.uint32(9)) | np.uint32(
                0x3F800000
            )
            flt = jax.lax.bitcast_convert_type(fbits, jnp.float32) - np.float32(1.0)
            u = jnp.maximum(flt, _TINY)
            val = x - jnp.log(-jnp.log(u))

            better = val > rv_ref[:, cs]
            rv_ref[:, cs] = jnp.where(better, val, rv_ref[:, cs])
            ri_ref[:, cs] = jnp.where(better, f42.astype(jnp.int32), ri_ref[:, cs])
        return 0

    jax.lax.fori_loop(0, _NBLK, step, 0, unroll=2)

    rv = rv_ref[...]
    col = (
        ri_ref[...]
        - jnp.int32(_KS1)
        - jax.lax.broadcasted_iota(jnp.int32, (_B, _BC), 0) * _V
    )
    bm = jnp.max(rv, axis=1, keepdims=True)
    bi = jnp.min(jnp.where(rv == bm, col, _BIG_IDX), axis=1, keepdims=True)
    out_ref[...] = bi


def kernel(logits):
    tail = jax.lax.slice(logits, (0, _V - _BC), (_B, _V))
    out = pl.pallas_call(
        _body,
        in_specs=[
            pl.BlockSpec(memory_space=pl.ANY),
            pl.BlockSpec(memory_space=pl.ANY),
        ],
        out_specs=pl.BlockSpec((_B, 1), memory_space=pltpu.VMEM),
        out_shape=jax.ShapeDtypeStruct((_B, 1), jnp.int32),
        scratch_shapes=[
            pltpu.VMEM((3, _B, _BC), jnp.float32),
            pltpu.VMEM((_B, _BC), jnp.int32),
            pltpu.VMEM((_B, _BC), jnp.float32),
            pltpu.VMEM((_B, _BC), jnp.int32),
            pltpu.SemaphoreType.DMA((3,)),
        ],
    )(logits, tail)
    return out.reshape(_B)
